# baseline (speedup 1.0000x reference)
import jax
import jax.numpy as jnp
from jax.experimental import pallas as pl

_B, _S, _D, _H = 16, 8192, 128, 4


def _dnc_body(mem_ref, ww_ref, wv_ref, ev_ref, keys_ref, st_ref, out_ref):
    mem = mem_ref[0]
    ww = ww_ref[0].reshape(_S, 1)
    wv = wv_ref[0]
    ev = ev_ref[0]
    keys = keys_ref[0]
    st = st_ref[0]

    updated = mem + ww * (wv - mem * ev)

    dot = jnp.dot(updated, keys, preferred_element_type=jnp.float32)
    mem_norm = jnp.sqrt(jnp.sum(updated * updated, axis=1, keepdims=True))
    key_norm = jnp.sqrt(jnp.sum(keys * keys, axis=0, keepdims=True))
    scores = dot / (mem_norm * key_norm + 1e-8) * st

    m = jnp.max(scores, axis=0, keepdims=True)
    e = jnp.exp(scores - m)
    w = e / jnp.sum(e, axis=0, keepdims=True)

    out_ref[0, :, 0:_D] = updated
    out_ref[0, :, _D:_D + _H] = w


def kernel(memory_matrix, write_weight, write_vector, erase_vector, keys, strengths):
    return pl.pallas_call(
        _dnc_body,
        grid=(_B,),
        in_specs=[
            pl.BlockSpec((1, _S, _D), lambda b: (b, 0, 0)),
            pl.BlockSpec((1, 1, _S), lambda b: (b, 0, 0)),
            pl.BlockSpec((1, 1, _D), lambda b: (b, 0, 0)),
            pl.BlockSpec((1, 1, _D), lambda b: (b, 0, 0)),
            pl.BlockSpec((1, _D, _H), lambda b: (b, 0, 0)),
            pl.BlockSpec((1, 1, _H), lambda b: (b, 0, 0)),
        ],
        out_specs=pl.BlockSpec((1, _S, _D + _H), lambda b: (b, 0, 0)),
        out_shape=jax.ShapeDtypeStruct((_B, _S, _D + _H), jnp.float32),
    )(
        memory_matrix,
        write_weight[:, None, :],
        write_vector[:, None, :],
        erase_vector[:, None, :],
        keys,
        strengths[:, None, :],
    )
